# Initial kernel scaffold; baseline (speedup 1.0000x reference)
import jax
import jax.numpy as jnp
from jax.experimental import pallas as pl


def kernel(style_ids, color_ids, occasion_ids, style_table, color_table, occasion_table, W1, b1, W2, b2, W3, b3):
    raise NotImplementedError("write your pallas kernel here")


# speedup vs baseline: 255.6897x; 255.6897x over previous
import functools

import jax
import jax.numpy as jnp
from jax import lax
from jax.experimental import pallas as pl
from jax.experimental.pallas import tpu as pltpu

_L = 200
_D = 256
_NBINS = 48
_TILE = 512


def _fused_body(s_ref, c_ref, o_ref, wf_ref, b1_ref, w2_ref, b2_ref,
                w3_ref, b3_ref, out_ref):
    t = s_ref.shape[0]
    lane = lax.broadcasted_iota(jnp.int32, (t, _NBINS), 1)
    counts = jnp.zeros((t, _NBINS), jnp.float32)
    for base, ref, vocab in ((0, s_ref, 20), (20, c_ref, 15), (35, o_ref, 10)):
        ids = ref[...]
        for v in range(vocab):
            col = jnp.sum((ids == v).astype(jnp.float32), axis=1,
                          keepdims=True)
            counts = jnp.where(lane == base + v, col, counts)
    h = jnp.maximum(
        jnp.dot(counts, wf_ref[...], preferred_element_type=jnp.float32)
        + b1_ref[...], 0.0)
    h = jnp.maximum(
        jnp.dot(h, w2_ref[...], preferred_element_type=jnp.float32)
        + b2_ref[...], 0.0)
    out_ref[...] = (
        jnp.dot(h, w3_ref[...], preferred_element_type=jnp.float32)
        + b3_ref[...])


def kernel(style_ids, color_ids, occasion_ids, style_table, color_table,
           occasion_table, W1, b1, W2, b2, W3, b3):
    b = style_ids.shape[0]
    q = style_table.shape[1]
    wf = jnp.concatenate([
        style_table @ W1[:q],
        color_table @ W1[q:2 * q],
        occasion_table @ W1[2 * q:3 * q],
    ], axis=0) * (1.0 / _L)
    wf = jnp.pad(wf, ((0, _NBINS - wf.shape[0]), (0, 0)))

    grid = (b // _TILE,)
    ids_spec = pl.BlockSpec((_TILE, _L), lambda i: (i, 0))
    w_spec = lambda shape: pl.BlockSpec(shape, lambda i: (0,) * len(shape))

    return pl.pallas_call(
        _fused_body,
        grid=grid,
        in_specs=[
            ids_spec, ids_spec, ids_spec,
            w_spec((_NBINS, _D)),
            w_spec((1, _D)),
            w_spec((_D, _D)),
            w_spec((1, _D)),
            w_spec((_D, _D)),
            w_spec((1, _D)),
        ],
        out_specs=pl.BlockSpec((_TILE, _D), lambda i: (i, 0)),
        out_shape=jax.ShapeDtypeStruct((b, _D), jnp.float32),
    )(style_ids, color_ids, occasion_ids, wf, b1.reshape(1, _D), W2,
      b2.reshape(1, _D), W3, b3.reshape(1, _D))
